# initial kernel scaffold (unmeasured)
import jax
import jax.numpy as jnp
from jax import lax
from jax.experimental import pallas as pl
from jax.experimental.pallas import tpu as pltpu

N_DEV = 8


def kernel(A, B):
    A = A.astype(jnp.bfloat16)
    B = B.astype(jnp.bfloat16)
    m_per, k = A.shape
    _, n = B.shape
    m_tile = 1024

    def body(a_ref, b_ref, out_ref, comm_ref, staging_ref,
             send_sems, recv_sems, credit_sems, copy_sem):
        my = lax.axis_index("i")
        left = lax.rem(my + (N_DEV - 1), N_DEV)
        right = lax.rem(my + 1, N_DEV)

        barrier_sem = pltpu.get_barrier_semaphore()
        for nbr in (left, right):
            pl.semaphore_signal(
                barrier_sem, inc=1,
                device_id=(nbr,), device_id_type=pl.DeviceIdType.MESH,
            )
        pl.semaphore_wait(barrier_sem, 2)

        def compute_store(origin, slot):
            for t in range(m_per // m_tile):
                rows = pl.ds(t * m_tile, m_tile)
                staging_ref[rows, :] = jnp.dot(
                    comm_ref[slot, rows, :], b_ref[...],
                    preferred_element_type=jnp.float32,
                ).astype(jnp.bfloat16)
            copy = pltpu.make_async_copy(
                staging_ref, out_ref.at[pl.ds(origin * m_per, m_per), :],
                copy_sem,
            )
            copy.start()
            copy.wait()

        comm_ref[0, :, :] = a_ref[...]
        compute_store(my, 0)

        for h in range(N_DEV - 1):
            s = h % 2
            r = (h + 1) % 2

            if h >= 1:
                pl.semaphore_wait(credit_sems.at[r], 1)

            rdma = pltpu.make_async_remote_copy(
                src_ref=comm_ref.at[s],
                dst_ref=comm_ref.at[r],
                send_sem=send_sems.at[s],
                recv_sem=recv_sems.at[r],
                device_id=(right,),
                device_id_type=pl.DeviceIdType.MESH,
            )
            rdma.start()
            rdma.wait()

            if h <= N_DEV - 3:
                pl.semaphore_signal(
                    credit_sems.at[s], inc=1,
                    device_id=(left,), device_id_type=pl.DeviceIdType.MESH,
                )

            origin = lax.rem(my + (N_DEV - 1 - h), N_DEV)
            compute_store(origin, r)

    out_shape = jax.ShapeDtypeStruct((N_DEV * m_per, n), jnp.bfloat16)
    return pl.pallas_call(
        body,
        out_shape=out_shape,
        in_specs=[
            pl.BlockSpec(memory_space=pltpu.MemorySpace.VMEM),
            pl.BlockSpec(memory_space=pltpu.MemorySpace.VMEM),
        ],
        out_specs=pl.BlockSpec(memory_space=pltpu.MemorySpace.HBM),
        scratch_shapes=[
            pltpu.VMEM((2, m_per, k), jnp.bfloat16),
            pltpu.VMEM((m_per, n), jnp.bfloat16),
            pltpu.SemaphoreType.DMA((2,)),
            pltpu.SemaphoreType.DMA((2,)),
            pltpu.SemaphoreType.REGULAR((2,)),
            pltpu.SemaphoreType.DMA,
        ],
        compiler_params=pltpu.CompilerParams(collective_id=0),
    )(A, B)


# baseline (device time: 2267829 ns/iter reference)
import jax
import jax.numpy as jnp
from jax import lax
from jax.experimental import pallas as pl
from jax.experimental.pallas import tpu as pltpu

N_DEV = 8


def kernel(A, B):
    A = A.astype(jnp.bfloat16)
    B = B.astype(jnp.bfloat16)
    m_per, k = A.shape
    _, n = B.shape
    m_tile = 256

    def body(a_ref, b_ref, out_ref, comm_ref, staging_ref,
             send_sems, recv_sems, credit_sems, copy_sem):
        my = lax.axis_index("i")
        left = lax.rem(my + (N_DEV - 1), N_DEV)
        right = lax.rem(my + 1, N_DEV)

        barrier_sem = pltpu.get_barrier_semaphore()
        for nbr in (left, right):
            pl.semaphore_signal(
                barrier_sem, inc=1,
                device_id=(nbr,), device_id_type=pl.DeviceIdType.MESH,
            )
        pl.semaphore_wait(barrier_sem, 2)

        def compute_store(origin, slot):
            def tile_body(t, _):
                rows = pl.ds(t * m_tile, m_tile)
                acc = jnp.dot(
                    comm_ref[slot, rows, :], b_ref[...],
                    preferred_element_type=jnp.float32,
                )
                staging_ref[...] = acc.astype(jnp.bfloat16)
                copy = pltpu.make_async_copy(
                    staging_ref,
                    out_ref.at[pl.ds(origin * m_per + t * m_tile, m_tile), :],
                    copy_sem,
                )
                copy.start()
                copy.wait()
                return 0
            lax.fori_loop(0, m_per // m_tile, tile_body, 0)

        load = pltpu.make_async_copy(a_ref, comm_ref.at[0], copy_sem)
        load.start()
        load.wait()
        compute_store(my, 0)

        for h in range(N_DEV - 1):
            s = h % 2
            r = (h + 1) % 2

            if h >= 1:
                pl.semaphore_wait(credit_sems.at[r], 1)

            rdma = pltpu.make_async_remote_copy(
                src_ref=comm_ref.at[s],
                dst_ref=comm_ref.at[r],
                send_sem=send_sems.at[s],
                recv_sem=recv_sems.at[r],
                device_id=(right,),
                device_id_type=pl.DeviceIdType.MESH,
            )
            rdma.start()
            rdma.wait()

            if h <= N_DEV - 3:
                pl.semaphore_signal(
                    credit_sems.at[s], inc=1,
                    device_id=(left,), device_id_type=pl.DeviceIdType.MESH,
                )

            origin = lax.rem(my + (N_DEV - 1 - h), N_DEV)
            compute_store(origin, r)

    out_shape = jax.ShapeDtypeStruct((N_DEV * m_per, n), jnp.bfloat16)
    return pl.pallas_call(
        body,
        out_shape=out_shape,
        in_specs=[
            pl.BlockSpec(memory_space=pltpu.MemorySpace.HBM),
            pl.BlockSpec(memory_space=pltpu.MemorySpace.VMEM),
        ],
        out_specs=pl.BlockSpec(memory_space=pltpu.MemorySpace.HBM),
        scratch_shapes=[
            pltpu.VMEM((2, m_per, k), jnp.bfloat16),
            pltpu.VMEM((m_tile, n), jnp.bfloat16),
            pltpu.SemaphoreType.DMA((2,)),
            pltpu.SemaphoreType.DMA((2,)),
            pltpu.SemaphoreType.REGULAR((2,)),
            pltpu.SemaphoreType.DMA,
        ],
        compiler_params=pltpu.CompilerParams(
            collective_id=0,
            vmem_limit_bytes=60 * 1024 * 1024,
        ),
    )(A, B)


# device time: 1592827 ns/iter; 1.4238x vs baseline; 1.4238x over previous
import jax
import jax.numpy as jnp
from jax import lax
from jax.experimental import pallas as pl
from jax.experimental.pallas import tpu as pltpu

N_DEV = 8


def kernel(A, B):
    A = A.astype(jnp.bfloat16)
    B = B.astype(jnp.bfloat16)
    m_per, k = A.shape
    _, n = B.shape
    m_tile = 256

    def body(a_ref, b_ref, out_ref, comm_ref, staging_ref,
             send_sems, recv_sems, credit_sems, copy_sem):
        my = lax.axis_index("i")
        left = lax.rem(my + (N_DEV - 1), N_DEV)
        right = lax.rem(my + 1, N_DEV)

        barrier_sem = pltpu.get_barrier_semaphore()
        for nbr in (left, right):
            pl.semaphore_signal(
                barrier_sem, inc=1,
                device_id=(nbr,), device_id_type=pl.DeviceIdType.MESH,
            )
        pl.semaphore_wait(barrier_sem, 2)

        def compute_store(origin, slot):
            def tile_body(t, _):
                rows = pl.ds(t * m_tile, m_tile)
                acc = jnp.dot(
                    comm_ref[slot, rows, :], b_ref[...],
                    preferred_element_type=jnp.float32,
                )
                staging_ref[...] = acc.astype(jnp.bfloat16)
                copy = pltpu.make_async_copy(
                    staging_ref,
                    out_ref.at[pl.ds(origin * m_per + t * m_tile, m_tile), :],
                    copy_sem,
                )
                copy.start()
                copy.wait()
                return 0
            lax.fori_loop(0, m_per // m_tile, tile_body, 0)

        load = pltpu.make_async_copy(a_ref, comm_ref.at[0], copy_sem)
        load.start()
        load.wait()

        for h in range(N_DEV - 1):
            s = h % 2
            r = (h + 1) % 2

            if h >= 1:
                pl.semaphore_wait(credit_sems.at[r], 1)

            rdma = pltpu.make_async_remote_copy(
                src_ref=comm_ref.at[s],
                dst_ref=comm_ref.at[r],
                send_sem=send_sems.at[s],
                recv_sem=recv_sems.at[r],
                device_id=(right,),
                device_id_type=pl.DeviceIdType.MESH,
            )
            rdma.start()

            origin = lax.rem(my + (N_DEV - h), N_DEV)
            compute_store(origin, s)

            rdma.wait()

            if h <= N_DEV - 3:
                pl.semaphore_signal(
                    credit_sems.at[s], inc=1,
                    device_id=(left,), device_id_type=pl.DeviceIdType.MESH,
                )

        compute_store(lax.rem(my + 1, N_DEV), (N_DEV - 1) % 2)

    out_shape = jax.ShapeDtypeStruct((N_DEV * m_per, n), jnp.bfloat16)
    return pl.pallas_call(
        body,
        out_shape=out_shape,
        in_specs=[
            pl.BlockSpec(memory_space=pltpu.MemorySpace.HBM),
            pl.BlockSpec(memory_space=pltpu.MemorySpace.VMEM),
        ],
        out_specs=pl.BlockSpec(memory_space=pltpu.MemorySpace.HBM),
        scratch_shapes=[
            pltpu.VMEM((2, m_per, k), jnp.bfloat16),
            pltpu.VMEM((m_tile, n), jnp.bfloat16),
            pltpu.SemaphoreType.DMA((2,)),
            pltpu.SemaphoreType.DMA((2,)),
            pltpu.SemaphoreType.REGULAR((2,)),
            pltpu.SemaphoreType.DMA,
        ],
        compiler_params=pltpu.CompilerParams(
            collective_id=0,
            vmem_limit_bytes=60 * 1024 * 1024,
        ),
    )(A, B)


# device time: 1002867 ns/iter; 2.2613x vs baseline; 1.5883x over previous
import jax
import jax.numpy as jnp
from jax import lax
from jax.experimental import pallas as pl
from jax.experimental.pallas import tpu as pltpu

N_DEV = 8


def kernel(A, B):
    A = A.astype(jnp.bfloat16)
    B = B.astype(jnp.bfloat16)
    m_per, k = A.shape
    _, n = B.shape
    half = m_per // 2
    m_tile = 256

    def body(a_ref, b_ref, out_ref, comm_cw, comm_ccw, staging_ref,
             cw_send_sems, cw_recv_sems, ccw_send_sems, ccw_recv_sems,
             cw_credit, ccw_credit, copy_sem):
        my = lax.axis_index("i")
        left = lax.rem(my + (N_DEV - 1), N_DEV)
        right = lax.rem(my + 1, N_DEV)

        barrier_sem = pltpu.get_barrier_semaphore()
        for nbr in (left, right):
            pl.semaphore_signal(
                barrier_sem, inc=1,
                device_id=(nbr,), device_id_type=pl.DeviceIdType.MESH,
            )
        pl.semaphore_wait(barrier_sem, 2)

        def compute_store(origin, half_idx, comm_ref, slot):
            base = origin * m_per + half_idx * half

            def tile_body(t, _):
                rows = pl.ds(t * m_tile, m_tile)
                acc = jnp.dot(
                    comm_ref[slot, rows, :], b_ref[...],
                    preferred_element_type=jnp.float32,
                )
                staging_ref[...] = acc.astype(jnp.bfloat16)
                copy = pltpu.make_async_copy(
                    staging_ref,
                    out_ref.at[pl.ds(base + t * m_tile, m_tile), :],
                    copy_sem,
                )
                copy.start()
                copy.wait()
                return 0
            lax.fori_loop(0, half // m_tile, tile_body, 0)

        load_top = pltpu.make_async_copy(
            a_ref.at[pl.ds(0, half), :], comm_cw.at[0], copy_sem)
        load_top.start()
        load_top.wait()
        load_bot = pltpu.make_async_copy(
            a_ref.at[pl.ds(half, half), :], comm_ccw.at[0], copy_sem)
        load_bot.start()
        load_bot.wait()

        for h in range(N_DEV - 1):
            s = h % 2
            r = (h + 1) % 2

            if h >= 1:
                pl.semaphore_wait(cw_credit.at[r], 1)
                pl.semaphore_wait(ccw_credit.at[r], 1)

            rdma_cw = pltpu.make_async_remote_copy(
                src_ref=comm_cw.at[s],
                dst_ref=comm_cw.at[r],
                send_sem=cw_send_sems.at[s],
                recv_sem=cw_recv_sems.at[r],
                device_id=(right,),
                device_id_type=pl.DeviceIdType.MESH,
            )
            rdma_cw.start()
            rdma_ccw = pltpu.make_async_remote_copy(
                src_ref=comm_ccw.at[s],
                dst_ref=comm_ccw.at[r],
                send_sem=ccw_send_sems.at[s],
                recv_sem=ccw_recv_sems.at[r],
                device_id=(left,),
                device_id_type=pl.DeviceIdType.MESH,
            )
            rdma_ccw.start()

            compute_store(lax.rem(my + (N_DEV - h), N_DEV), 0, comm_cw, s)
            compute_store(lax.rem(my + h, N_DEV), 1, comm_ccw, s)

            rdma_cw.wait()
            rdma_ccw.wait()

            if h <= N_DEV - 3:
                pl.semaphore_signal(
                    cw_credit.at[s], inc=1,
                    device_id=(left,), device_id_type=pl.DeviceIdType.MESH,
                )
                pl.semaphore_signal(
                    ccw_credit.at[s], inc=1,
                    device_id=(right,), device_id_type=pl.DeviceIdType.MESH,
                )

        last = (N_DEV - 1) % 2
        compute_store(lax.rem(my + 1, N_DEV), 0, comm_cw, last)
        compute_store(lax.rem(my + (N_DEV - 1), N_DEV), 1, comm_ccw, last)

    out_shape = jax.ShapeDtypeStruct((N_DEV * m_per, n), jnp.bfloat16)
    return pl.pallas_call(
        body,
        out_shape=out_shape,
        in_specs=[
            pl.BlockSpec(memory_space=pltpu.MemorySpace.HBM),
            pl.BlockSpec(memory_space=pltpu.MemorySpace.VMEM),
        ],
        out_specs=pl.BlockSpec(memory_space=pltpu.MemorySpace.HBM),
        scratch_shapes=[
            pltpu.VMEM((2, half, k), jnp.bfloat16),
            pltpu.VMEM((2, half, k), jnp.bfloat16),
            pltpu.VMEM((m_tile, n), jnp.bfloat16),
            pltpu.SemaphoreType.DMA((2,)),
            pltpu.SemaphoreType.DMA((2,)),
            pltpu.SemaphoreType.DMA((2,)),
            pltpu.SemaphoreType.DMA((2,)),
            pltpu.SemaphoreType.REGULAR((2,)),
            pltpu.SemaphoreType.REGULAR((2,)),
            pltpu.SemaphoreType.DMA,
        ],
        compiler_params=pltpu.CompilerParams(
            collective_id=0,
            vmem_limit_bytes=60 * 1024 * 1024,
        ),
    )(A, B)


# device time: 947400 ns/iter; 2.3937x vs baseline; 1.0585x over previous
import jax
import jax.numpy as jnp
from jax import lax
from jax.experimental import pallas as pl
from jax.experimental.pallas import tpu as pltpu

N_DEV = 8


def kernel(A, B):
    A = A.astype(jnp.bfloat16)
    B = B.astype(jnp.bfloat16)
    m_per, k = A.shape
    _, n = B.shape
    half = m_per // 2
    m_tile = 256

    def body(a_ref, b_ref, out_ref, comm_cw, comm_ccw, staging_ref,
             cw_send_sems, cw_recv_sems, ccw_send_sems, ccw_recv_sems,
             cw_credit, ccw_credit, copy_sems, copy_sem):
        my = lax.axis_index("i")
        left = lax.rem(my + (N_DEV - 1), N_DEV)
        right = lax.rem(my + 1, N_DEV)

        barrier_sem = pltpu.get_barrier_semaphore()
        for nbr in (left, right):
            pl.semaphore_signal(
                barrier_sem, inc=1,
                device_id=(nbr,), device_id_type=pl.DeviceIdType.MESH,
            )
        pl.semaphore_wait(barrier_sem, 2)

        def compute_store(origin, half_idx, comm_ref, slot):
            base = origin * m_per + half_idx * half

            def tile_body(t, _):
                b_slot = lax.rem(t, 2)
                copy = pltpu.make_async_copy(
                    staging_ref.at[b_slot],
                    out_ref.at[pl.ds(base + t * m_tile, m_tile), :],
                    copy_sems.at[b_slot],
                )

                @pl.when(t >= 2)
                def _():
                    copy.wait()

                acc = jnp.dot(
                    comm_ref[slot, pl.ds(t * m_tile, m_tile), :], b_ref[...],
                    preferred_element_type=jnp.float32,
                )
                staging_ref[b_slot, :, :] = acc.astype(jnp.bfloat16)
                copy.start()
                return 0
            n_tiles = half // m_tile
            lax.fori_loop(0, n_tiles, tile_body, 0)

            for j in (0, 1):
                t_last = n_tiles - 2 + j
                pltpu.make_async_copy(
                    staging_ref.at[j],
                    out_ref.at[pl.ds(base + t_last * m_tile, m_tile), :],
                    copy_sems.at[j],
                ).wait()

        load_top = pltpu.make_async_copy(
            a_ref.at[pl.ds(0, half), :], comm_cw.at[0], copy_sem)
        load_top.start()
        load_top.wait()
        load_bot = pltpu.make_async_copy(
            a_ref.at[pl.ds(half, half), :], comm_ccw.at[0], copy_sem)
        load_bot.start()
        load_bot.wait()

        for h in range(N_DEV - 1):
            s = h % 2
            r = (h + 1) % 2

            if h >= 1:
                pl.semaphore_wait(cw_credit.at[r], 1)
                pl.semaphore_wait(ccw_credit.at[r], 1)

            rdma_cw = pltpu.make_async_remote_copy(
                src_ref=comm_cw.at[s],
                dst_ref=comm_cw.at[r],
                send_sem=cw_send_sems.at[s],
                recv_sem=cw_recv_sems.at[r],
                device_id=(right,),
                device_id_type=pl.DeviceIdType.MESH,
            )
            rdma_cw.start()
            rdma_ccw = pltpu.make_async_remote_copy(
                src_ref=comm_ccw.at[s],
                dst_ref=comm_ccw.at[r],
                send_sem=ccw_send_sems.at[s],
                recv_sem=ccw_recv_sems.at[r],
                device_id=(left,),
                device_id_type=pl.DeviceIdType.MESH,
            )
            rdma_ccw.start()

            compute_store(lax.rem(my + (N_DEV - h), N_DEV), 0, comm_cw, s)
            compute_store(lax.rem(my + h, N_DEV), 1, comm_ccw, s)

            rdma_cw.wait()
            rdma_ccw.wait()

            if h <= N_DEV - 3:
                pl.semaphore_signal(
                    cw_credit.at[s], inc=1,
                    device_id=(left,), device_id_type=pl.DeviceIdType.MESH,
                )
                pl.semaphore_signal(
                    ccw_credit.at[s], inc=1,
                    device_id=(right,), device_id_type=pl.DeviceIdType.MESH,
                )

        last = (N_DEV - 1) % 2
        compute_store(lax.rem(my + 1, N_DEV), 0, comm_cw, last)
        compute_store(lax.rem(my + (N_DEV - 1), N_DEV), 1, comm_ccw, last)

    out_shape = jax.ShapeDtypeStruct((N_DEV * m_per, n), jnp.bfloat16)
    return pl.pallas_call(
        body,
        out_shape=out_shape,
        in_specs=[
            pl.BlockSpec(memory_space=pltpu.MemorySpace.HBM),
            pl.BlockSpec(memory_space=pltpu.MemorySpace.VMEM),
        ],
        out_specs=pl.BlockSpec(memory_space=pltpu.MemorySpace.HBM),
        scratch_shapes=[
            pltpu.VMEM((2, half, k), jnp.bfloat16),
            pltpu.VMEM((2, half, k), jnp.bfloat16),
            pltpu.VMEM((2, m_tile, n), jnp.bfloat16),
            pltpu.SemaphoreType.DMA((2,)),
            pltpu.SemaphoreType.DMA((2,)),
            pltpu.SemaphoreType.DMA((2,)),
            pltpu.SemaphoreType.DMA((2,)),
            pltpu.SemaphoreType.REGULAR((2,)),
            pltpu.SemaphoreType.REGULAR((2,)),
            pltpu.SemaphoreType.DMA((2,)),
            pltpu.SemaphoreType.DMA,
        ],
        compiler_params=pltpu.CompilerParams(
            collective_id=0,
            vmem_limit_bytes=60 * 1024 * 1024,
        ),
    )(A, B)
